# baseline (speedup 1.0000x reference)
import jax
import jax.numpy as jnp
from jax.experimental import pallas as pl
from jax.experimental.pallas import tpu as pltpu

B, N, D_MODEL, LAP_DIM = 8, 2048, 128, 8
C = 2
R = N // C
NC = B * C


def _fused_kernel(nf_hbm, evt_hbm, wt_ref, g_ref, out_hbm,
                  nf_v, ev_v, out_v, g_v, nf_sems, ev_sems, out_sems, g_sem):
    for i in range(NC):
        b, c = divmod(i, C)
        pltpu.make_async_copy(
            nf_hbm.at[b, pl.ds(c * R, R), :], nf_v.at[i], nf_sems.at[i]
        ).start()
    for b in range(B):
        pltpu.make_async_copy(evt_hbm.at[b], ev_v.at[b], ev_sems.at[b]).start()
    g_v[...] = jnp.broadcast_to(g_ref[0], (B, D_MODEL))
    pltpu.make_async_copy(g_v, out_hbm.at[0], g_sem).start()
    wt = wt_ref[...]
    for i in range(NC):
        b, c = divmod(i, C)
        if c == 0:
            pltpu.make_async_copy(evt_hbm.at[b], ev_v.at[b], ev_sems.at[b]).wait()
        pltpu.make_async_copy(
            nf_hbm.at[b, pl.ds(c * R, R), :], nf_v.at[i], nf_sems.at[i]
        ).wait()
        lap = jax.lax.dot_general(
            ev_v[b, :, pl.ds(c * R, R)], wt, (((0,), (0,)), ((), ())),
            preferred_element_type=jnp.float32)
        out_v[i] = nf_v[i] + lap
        pltpu.make_async_copy(
            out_v.at[i], out_hbm.at[pl.ds(1 + c * R, R), b, :], out_sems.at[i]
        ).start()
    for i in range(NC):
        b, c = divmod(i, C)
        pltpu.make_async_copy(
            out_v.at[i], out_hbm.at[pl.ds(1 + c * R, R), b, :], out_sems.at[i]
        ).wait()
    pltpu.make_async_copy(g_v, out_hbm.at[0], g_sem).wait()


def kernel(adj, node_feats, eigvec, W_lap, graph_tok):
    b, n, _ = adj.shape
    d = node_feats.shape[-1]
    lap_dim = eigvec.shape[-1]
    ev_t = jnp.transpose(eigvec, (0, 2, 1))
    w_t = jnp.transpose(W_lap, (1, 0))
    out_t = pl.pallas_call(
        _fused_kernel,
        in_specs=[
            pl.BlockSpec(memory_space=pl.ANY),
            pl.BlockSpec(memory_space=pl.ANY),
            pl.BlockSpec(w_t.shape, lambda: (0, 0)),
            pl.BlockSpec(graph_tok.shape, lambda: (0, 0, 0)),
        ],
        out_specs=pl.BlockSpec(memory_space=pl.ANY),
        out_shape=jax.ShapeDtypeStruct((1 + n, b, d), jnp.float32),
        scratch_shapes=[
            pltpu.MemorySpace.VMEM((NC, R, d), jnp.float32),
            pltpu.MemorySpace.VMEM((b, lap_dim, n), jnp.float32),
            pltpu.MemorySpace.VMEM((NC, R, d), jnp.float32),
            pltpu.MemorySpace.VMEM((b, d), jnp.float32),
            pltpu.SemaphoreType.DMA((NC,)),
            pltpu.SemaphoreType.DMA((b,)),
            pltpu.SemaphoreType.DMA((NC,)),
            pltpu.SemaphoreType.DMA,
        ],
    )(node_feats, ev_t, w_t, graph_tok)
    seq = jnp.transpose(out_t, (1, 0, 2))
    pad_mask = jnp.zeros((b, 1 + n), dtype=bool)
    return seq, pad_mask


# speedup vs baseline: 1.2788x; 1.2788x over previous
import jax
import jax.numpy as jnp
from jax.experimental import pallas as pl
from jax.experimental.pallas import tpu as pltpu

B, N, D_MODEL, LAP_DIM = 8, 2048, 128, 8
C = 1
R = N // C
NC = B * C


def _fused_kernel(nf_hbm, evt_hbm, wt_ref, g_ref, out_hbm,
                  nf_v, ev_v, out_v, g_v, nf_sems, ev_sems, out_sems, g_sem):
    for i in range(NC):
        b, c = divmod(i, C)
        pltpu.make_async_copy(
            nf_hbm.at[b, pl.ds(c * R, R), :], nf_v.at[i], nf_sems.at[i]
        ).start()
    for b in range(B):
        pltpu.make_async_copy(evt_hbm.at[b], ev_v.at[b], ev_sems.at[b]).start()
    g_v[...] = jnp.broadcast_to(g_ref[0], (B, D_MODEL))
    pltpu.make_async_copy(g_v, out_hbm.at[0], g_sem).start()
    wt = wt_ref[...]
    for i in range(NC):
        b, c = divmod(i, C)
        if c == 0:
            pltpu.make_async_copy(evt_hbm.at[b], ev_v.at[b], ev_sems.at[b]).wait()
        pltpu.make_async_copy(
            nf_hbm.at[b, pl.ds(c * R, R), :], nf_v.at[i], nf_sems.at[i]
        ).wait()
        lap = jax.lax.dot_general(
            ev_v[b, :, pl.ds(c * R, R)], wt, (((0,), (0,)), ((), ())),
            preferred_element_type=jnp.float32)
        out_v[i] = nf_v[i] + lap
        pltpu.make_async_copy(
            out_v.at[i], out_hbm.at[pl.ds(1 + c * R, R), b, :], out_sems.at[i]
        ).start()
    for i in range(NC):
        b, c = divmod(i, C)
        pltpu.make_async_copy(
            out_v.at[i], out_hbm.at[pl.ds(1 + c * R, R), b, :], out_sems.at[i]
        ).wait()
    pltpu.make_async_copy(g_v, out_hbm.at[0], g_sem).wait()


def kernel(adj, node_feats, eigvec, W_lap, graph_tok):
    b, n, _ = adj.shape
    d = node_feats.shape[-1]
    lap_dim = eigvec.shape[-1]
    ev_t = jnp.transpose(eigvec, (0, 2, 1))
    w_t = jnp.transpose(W_lap, (1, 0))
    out_t = pl.pallas_call(
        _fused_kernel,
        in_specs=[
            pl.BlockSpec(memory_space=pl.ANY),
            pl.BlockSpec(memory_space=pl.ANY),
            pl.BlockSpec(w_t.shape, lambda: (0, 0)),
            pl.BlockSpec(graph_tok.shape, lambda: (0, 0, 0)),
        ],
        out_specs=pl.BlockSpec(memory_space=pl.ANY),
        out_shape=jax.ShapeDtypeStruct((1 + n, b, d), jnp.float32),
        scratch_shapes=[
            pltpu.MemorySpace.VMEM((NC, R, d), jnp.float32),
            pltpu.MemorySpace.VMEM((b, lap_dim, n), jnp.float32),
            pltpu.MemorySpace.VMEM((NC, R, d), jnp.float32),
            pltpu.MemorySpace.VMEM((b, d), jnp.float32),
            pltpu.SemaphoreType.DMA((NC,)),
            pltpu.SemaphoreType.DMA((b,)),
            pltpu.SemaphoreType.DMA((NC,)),
            pltpu.SemaphoreType.DMA,
        ],
    )(node_feats, ev_t, w_t, graph_tok)
    seq = jnp.transpose(out_t, (1, 0, 2))
    pad_mask = jnp.zeros((b, 1 + n), dtype=bool)
    return seq, pad_mask
